# baseline (speedup 1.0000x reference)
import functools

import jax
import jax.numpy as jnp
from jax import lax
from jax.experimental import pallas as pl
from jax.experimental.pallas import tpu as pltpu
from jax.experimental.pallas import tpu_sc as plsc


def _sc_scores(V, D, B, K, chunk):
  info = plsc.get_sparse_core_info()
  NC, NS = info.num_cores, info.num_subcores
  NW = NC * NS
  P = K + 1
  assert B % NW == 0
  per_w = B // NW
  assert per_w % chunk == 0 and chunk % 16 == 0
  n_chunks = per_w // chunk
  n_groups = chunk // 16
  G = P * chunk
  mesh = plsc.VectorSubcoreMesh(core_axis_name="c", subcore_axis_name="s")

  @functools.partial(
      pl.kernel,
      mesh=mesh,
      compiler_params=pltpu.CompilerParams(needs_layout_passes=False),
      out_type=jax.ShapeDtypeStruct((P * B,), jnp.float32),
      scratch_types=[
          pltpu.VMEM((per_w,), jnp.int32),
          pltpu.VMEM((n_chunks * G,), jnp.int32),
          pltpu.VMEM((2, chunk, D), jnp.float32),
          pltpu.VMEM((2, G, D), jnp.float32),
          pltpu.VMEM((P * per_w,), jnp.float32),
          pltpu.SemaphoreType.DMA,
      ],
  )
  def k(in_hbm, out_hbm, cen_ids, rest_ids, scores_hbm,
        cidx_v, ridx_v, cen_v, oth_v, sc_v, sem):
    wid = lax.axis_index("s") * NC + lax.axis_index("c")
    base_w = wid * per_w
    lanes = lax.iota(jnp.int32, 16)

    pltpu.sync_copy(cen_ids.at[pl.ds(base_w, per_w)], cidx_v)
    pltpu.sync_copy(rest_ids.at[pl.ds(wid * n_chunks * G, n_chunks * G)],
                    ridx_v)


    def fire(c, buf):
      pltpu.async_copy(in_hbm.at[cidx_v.at[pl.ds(c * chunk, chunk)]],
                       cen_v.at[buf], sem)
      pltpu.async_copy(out_hbm.at[ridx_v.at[pl.ds(c * G, G)]],
                       oth_v.at[buf], sem)

    fire(0, 0)

    def chunk_body(c, carry):
      buf = lax.rem(c, 2)

      @pl.when(c + 1 < n_chunks)
      def _fire_next():
        fire(c + 1, lax.rem(c + 1, 2))

      pltpu.make_async_copy(in_hbm.at[pl.ds(0, chunk)], cen_v.at[buf],
                            sem).wait()
      pltpu.make_async_copy(out_hbm.at[pl.ds(0, G)], oth_v.at[buf],
                            sem).wait()

      @plsc.parallel_loop(0, n_groups)
      def group_body(g):
        row0 = g * 16
        svecs = [jnp.zeros((16,), jnp.float32) for _ in range(P)]
        for r in range(16):
          row = row0 + r
          accs = [None] * P
          for j in range(D // 16):
            cv = cen_v[buf, row, pl.ds(16 * j, 16)]
            for p in range(P):
              t = cv * oth_v[buf, p * chunk + row, pl.ds(16 * j, 16)]
              accs[p] = t if j == 0 else accs[p] + t
          for p in range(P):
            s = plsc.cumsum(accs[p])[15]
            svecs[p] = jnp.where(lanes == r, s, svecs[p])
        off = c * chunk + row0
        for p in range(P):
          sc_v[pl.ds(p * per_w + off, 16)] = svecs[p]

      return carry

    lax.fori_loop(0, n_chunks, chunk_body, 0)
    for p in range(P):
      pltpu.sync_copy(sc_v.at[pl.ds(p * per_w, per_w)],
                      scores_hbm.at[pl.ds(p * B + base_w, per_w)])

  return k


def _tc_loss_body(scores_ref, acc_ref, *, n_pos, n_neg):
  scores = scores_ref[...]
  pos = scores[0]
  neg = scores[1:]
  pos_terms = -jnp.log(jax.nn.sigmoid(pos) + 1e-08)
  neg_terms = -jnp.log(jax.nn.sigmoid(-neg) + 1e-08)
  acc_ref[...] = jnp.full((1, 1), jnp.sum(pos_terms) / n_pos +
                          jnp.sum(neg_terms) / n_neg)


def kernel(in_embed, out_embed, center_ids, context_ids, negative_ids,
           vocab_size):
  V, D = in_embed.shape
  B = center_ids.shape[0]
  K = negative_ids.shape[0]
  P = K + 1
  chunk = 64
  info = plsc.get_sparse_core_info()
  NW = info.num_cores * info.num_subcores
  per_w = B // NW
  n_chunks = per_w // chunk

  rest_ids = jnp.concatenate([context_ids, negative_ids.reshape(-1)])
  rids = rest_ids.reshape(P, NW, n_chunks, chunk).transpose(1, 2, 0, 3)
  scores = _sc_scores(V, D, B, K, chunk)(
      in_embed, out_embed, center_ids, rids.reshape(-1))
  scores = scores.reshape(P, B)

  acc = pl.pallas_call(
      functools.partial(_tc_loss_body, n_pos=float(B), n_neg=float(K * B)),
      out_shape=jax.ShapeDtypeStruct((1, 1), jnp.float32),
  )(scores)
  return acc[0, 0]


# speedup vs baseline: 1.1189x; 1.1189x over previous
import functools

import jax
import jax.numpy as jnp
from jax import lax
from jax.experimental import pallas as pl
from jax.experimental.pallas import tpu as pltpu
from jax.experimental.pallas import tpu_sc as plsc


def _sc_scores(V, D, B, K, chunk):
  info = plsc.get_sparse_core_info()
  NC, NS = info.num_cores, info.num_subcores
  NW = NC * NS
  P = K + 1
  assert B % NW == 0
  per_w = B // NW
  assert per_w % chunk == 0 and chunk % 16 == 0
  n_chunks = per_w // chunk
  n_groups = chunk // 16
  G = P * chunk
  mesh = plsc.VectorSubcoreMesh(core_axis_name="c", subcore_axis_name="s")

  @functools.partial(
      pl.kernel,
      mesh=mesh,
      compiler_params=pltpu.CompilerParams(needs_layout_passes=False),
      out_type=jax.ShapeDtypeStruct((P * B,), jnp.float32),
      scratch_types=[
          pltpu.VMEM((per_w,), jnp.int32),
          pltpu.VMEM((n_chunks * G,), jnp.int32),
          pltpu.VMEM((2, chunk, D), jnp.float32),
          pltpu.VMEM((2, G, D), jnp.float32),
          pltpu.VMEM((P * per_w,), jnp.float32),
          pltpu.SemaphoreType.DMA,
      ],
  )
  def k(in_hbm, out_hbm, cen_ids, rest_ids, scores_hbm,
        cidx_v, ridx_v, cen_v, oth_v, sc_v, sem):
    wid = lax.axis_index("s") * NC + lax.axis_index("c")
    base_w = wid * per_w
    lanes = lax.iota(jnp.int32, 16)

    pltpu.sync_copy(cen_ids.at[pl.ds(base_w, per_w)], cidx_v)
    pltpu.sync_copy(rest_ids.at[pl.ds(wid * n_chunks * G, n_chunks * G)],
                    ridx_v)


    def fire(c, buf):
      pltpu.async_copy(in_hbm.at[cidx_v.at[pl.ds(c * chunk, chunk)]],
                       cen_v.at[buf], sem)
      pltpu.async_copy(out_hbm.at[ridx_v.at[pl.ds(c * G, G)]],
                       oth_v.at[buf], sem)

    fire(0, 0)

    def chunk_body(c, carry):
      buf = lax.rem(c, 2)

      @pl.when(c + 1 < n_chunks)
      def _fire_next():
        fire(c + 1, lax.rem(c + 1, 2))

      pltpu.make_async_copy(in_hbm.at[pl.ds(0, chunk)], cen_v.at[buf],
                            sem).wait()
      pltpu.make_async_copy(out_hbm.at[pl.ds(0, G)], oth_v.at[buf],
                            sem).wait()

      pvec = lanes * per_w
      lt6 = lanes < P

      @plsc.parallel_loop(0, chunk, unroll=8)
      def row_body(row):
        accs = [None] * P
        for j in range(D // 16):
          cv = cen_v[buf, row, pl.ds(16 * j, 16)]
          for p in range(P):
            t = cv * oth_v[buf, p * chunk + row, pl.ds(16 * j, 16)]
            accs[p] = t if j == 0 else accs[p] + t
        svec = jnp.zeros((16,), jnp.float32)
        for p in range(P):
          s = plsc.cumsum(accs[p])[15]
          svec = jnp.where(lanes == p, s, svec)
        plsc.store_scatter(sc_v, [pvec + (c * chunk + row)], svec, mask=lt6)

      return carry

    lax.fori_loop(0, n_chunks, chunk_body, 0)
    for p in range(P):
      pltpu.sync_copy(sc_v.at[pl.ds(p * per_w, per_w)],
                      scores_hbm.at[pl.ds(p * B + base_w, per_w)])

  return k


def _tc_loss_body(scores_ref, acc_ref, *, n_pos, n_neg):
  scores = scores_ref[...]
  pos = scores[0]
  neg = scores[1:]
  pos_terms = -jnp.log(jax.nn.sigmoid(pos) + 1e-08)
  neg_terms = -jnp.log(jax.nn.sigmoid(-neg) + 1e-08)
  acc_ref[...] = jnp.full((1, 1), jnp.sum(pos_terms) / n_pos +
                          jnp.sum(neg_terms) / n_neg)


def kernel(in_embed, out_embed, center_ids, context_ids, negative_ids,
           vocab_size):
  V, D = in_embed.shape
  B = center_ids.shape[0]
  K = negative_ids.shape[0]
  P = K + 1
  chunk = 64
  info = plsc.get_sparse_core_info()
  NW = info.num_cores * info.num_subcores
  per_w = B // NW
  n_chunks = per_w // chunk

  rest_ids = jnp.concatenate([context_ids, negative_ids.reshape(-1)])
  rids = rest_ids.reshape(P, NW, n_chunks, chunk).transpose(1, 2, 0, 3)
  scores = _sc_scores(V, D, B, K, chunk)(
      in_embed, out_embed, center_ids, rids.reshape(-1))
  scores = scores.reshape(P, B)

  acc = pl.pallas_call(
      functools.partial(_tc_loss_body, n_pos=float(B), n_neg=float(K * B)),
      out_shape=jax.ShapeDtypeStruct((1, 1), jnp.float32),
  )(scores)
  return acc[0, 0]


# speedup vs baseline: 2.1608x; 1.9313x over previous
import functools

import jax
import jax.numpy as jnp
from jax import lax
from jax.experimental import pallas as pl
from jax.experimental.pallas import tpu as pltpu
from jax.experimental.pallas import tpu_sc as plsc


def _sc_scores(V, D, B, K, chunk):
  info = plsc.get_sparse_core_info()
  NC, NS = info.num_cores, info.num_subcores
  NW = NC * NS
  P = K + 1
  assert B % NW == 0
  per_w = B // NW
  assert per_w % chunk == 0 and chunk % 16 == 0
  n_chunks = per_w // chunk
  n_groups = chunk // 16
  G = P * chunk
  mesh = plsc.VectorSubcoreMesh(core_axis_name="c", subcore_axis_name="s")

  @functools.partial(
      pl.kernel,
      mesh=mesh,
      compiler_params=pltpu.CompilerParams(needs_layout_passes=False),
      out_type=jax.ShapeDtypeStruct((P * B,), jnp.float32),
      scratch_types=[
          pltpu.VMEM((per_w,), jnp.int32),
          pltpu.VMEM((n_chunks * G,), jnp.int32),
          pltpu.VMEM((2, chunk, D), jnp.float32),
          pltpu.VMEM((2, G, D), jnp.float32),
          pltpu.VMEM((P * per_w,), jnp.float32),
          pltpu.SemaphoreType.DMA,
      ],
  )
  def k(in_hbm, out_hbm, cen_ids, rest_ids, scores_hbm,
        cidx_v, ridx_v, cen_v, oth_v, sc_v, sem):
    wid = lax.axis_index("s") * NC + lax.axis_index("c")
    base_w = wid * per_w
    lanes = lax.iota(jnp.int32, 16)
    perms = [lanes ^ (1 << t) for t in range(4)]
    gd = lax.GatherDimensionNumbers(
        offset_dims=(), collapsed_slice_dims=(0,), start_index_map=(0,))

    def shuffle(v, pm):
      return lax.gather(v, pm[:, None], gd, slice_sizes=(1,),
                        mode=lax.GatherScatterMode.PROMISE_IN_BOUNDS)

    pltpu.sync_copy(cen_ids.at[pl.ds(base_w, per_w)], cidx_v)
    pltpu.sync_copy(rest_ids.at[pl.ds(wid * n_chunks * G, n_chunks * G)],
                    ridx_v)


    def fire(c, buf):
      pltpu.async_copy(in_hbm.at[cidx_v.at[pl.ds(c * chunk, chunk)]],
                       cen_v.at[buf], sem)
      pltpu.async_copy(out_hbm.at[ridx_v.at[pl.ds(c * G, G)]],
                       oth_v.at[buf], sem)

    fire(0, 0)

    def chunk_body(c, carry):
      buf = lax.rem(c, 2)

      @pl.when(c + 1 < n_chunks)
      def _fire_next():
        fire(c + 1, lax.rem(c + 1, 2))

      pltpu.make_async_copy(in_hbm.at[pl.ds(0, chunk)], cen_v.at[buf],
                            sem).wait()
      pltpu.make_async_copy(out_hbm.at[pl.ds(0, G)], oth_v.at[buf],
                            sem).wait()

      pvec = lanes * per_w
      lt6 = lanes < P

      @plsc.parallel_loop(0, chunk, unroll=8)
      def row_body(row):
        accs = [None] * P
        for j in range(D // 16):
          cv = cen_v[buf, row, pl.ds(16 * j, 16)]
          for p in range(P):
            t = cv * oth_v[buf, p * chunk + row, pl.ds(16 * j, 16)]
            accs[p] = t if j == 0 else accs[p] + t
        svec = jnp.zeros((16,), jnp.float32)
        for p in range(P):
          s = accs[p]
          for pm in perms:
            s = s + shuffle(s, pm)
          svec = jnp.where(lanes == p, s, svec)
        plsc.store_scatter(sc_v, [pvec + (c * chunk + row)], svec, mask=lt6)

      return carry

    lax.fori_loop(0, n_chunks, chunk_body, 0)
    for p in range(P):
      pltpu.sync_copy(sc_v.at[pl.ds(p * per_w, per_w)],
                      scores_hbm.at[pl.ds(p * B + base_w, per_w)])

  return k


def _tc_loss_body(scores_ref, acc_ref, *, n_pos, n_neg):
  scores = scores_ref[...]
  pos = scores[0]
  neg = scores[1:]
  pos_terms = -jnp.log(jax.nn.sigmoid(pos) + 1e-08)
  neg_terms = -jnp.log(jax.nn.sigmoid(-neg) + 1e-08)
  acc_ref[...] = jnp.full((1, 1), jnp.sum(pos_terms) / n_pos +
                          jnp.sum(neg_terms) / n_neg)


def kernel(in_embed, out_embed, center_ids, context_ids, negative_ids,
           vocab_size):
  V, D = in_embed.shape
  B = center_ids.shape[0]
  K = negative_ids.shape[0]
  P = K + 1
  chunk = 64
  info = plsc.get_sparse_core_info()
  NW = info.num_cores * info.num_subcores
  per_w = B // NW
  n_chunks = per_w // chunk

  rest_ids = jnp.concatenate([context_ids, negative_ids.reshape(-1)])
  rids = rest_ids.reshape(P, NW, n_chunks, chunk).transpose(1, 2, 0, 3)
  scores = _sc_scores(V, D, B, K, chunk)(
      in_embed, out_embed, center_ids, rids.reshape(-1))
  scores = scores.reshape(P, B)

  acc = pl.pallas_call(
      functools.partial(_tc_loss_body, n_pos=float(B), n_neg=float(K * B)),
      out_shape=jax.ShapeDtypeStruct((1, 1), jnp.float32),
  )(scores)
  return acc[0, 0]
